# baseline (speedup 1.0000x reference)
import functools

import numpy as np
import jax
import jax.numpy as jnp
from jax import lax
from jax.experimental import pallas as pl
from jax.experimental.pallas import tpu as pltpu
from jax.experimental.pallas import tpu_sc as plsc

ND = 2000
NM = 3000
N = ND + NM
HID = 64
B = 16384
ALPHA = 0.1
BETA = 0.1
LAMDA = 0.5
THETA1 = float(np.log(LAMDA / 1.0 + 1.0))
THETA2 = float(np.log(LAMDA / 2.0 + 1.0))
RB = 200
G = N // RB
GD = ND // RB

_f32 = jnp.float32


def _dot(a, b):
    return jnp.dot(a, b, preferred_element_type=_f32)


def _elu(x):
    return jnp.where(x > 0, x, jnp.exp(jnp.minimum(x, 0.0)) - 1.0)


def _layer_epilogue(agg, h0, mu, sig, gam, w, watt_t, watt_b, theta):
    h_agg = (1.0 - ALPHA) * agg + ALPHA * h0
    h_i = theta * _dot(h_agg, w) + (1.0 - theta) * h_agg
    qb = _dot(h_i, watt_b)
    e_mu = _elu(_dot(mu, watt_t) + qb)
    e_si = _elu(_dot(sig, watt_t) + qb)
    e_ga = _elu(_dot(gam, watt_t) + qb)
    m = jnp.maximum(jnp.maximum(e_mu, e_si), e_ga)
    x_mu = jnp.exp(e_mu - m)
    x_si = jnp.exp(e_si - m)
    x_ga = jnp.exp(e_ga - m)
    h_mom = (mu * x_mu + sig * x_si + gam * x_ga) / (x_mu + x_si + x_ga)
    out = (1.0 - BETA) * h_i + BETA * h_mom
    rm = jnp.max(out, axis=1, keepdims=True)
    e = jnp.exp(out - rm)
    return e / jnp.sum(e, axis=1, keepdims=True)


def _fused_body(adj, dsim, msim, topo, wdfcT, wmfcT, wdtT, wdfT, bdown,
                wfc0T, bfc0, wc1, w1t, w1b, wc2, w2t, w2b,
                w1T, b1, w2T, b2, wd1aT, wd1bT, bd1, wm1aT, wm1bT, bm1,
                wp, bp2, ab_o, feats_s, rhs_s, h1_s, mom_s):
    pi = pl.program_id(0)
    pj = pl.program_id(1)

    @pl.when(pi == 0)
    def _pass0():
        row0 = pj * RB

        def finish(f):
            x = (_dot(topo[...], wdtT[...]) + _dot(f, wdfT[...])
                 + bdown[...])
            h0 = jnp.maximum(_dot(x, wfc0T[...]) + bfc0[...], 0.0)
            feats_s[pl.ds(row0, RB), :] = f
            rhs_s[pl.ds(row0, RB), :] = jnp.concatenate(
                [h0, h0 * h0, h0 * h0 * h0], axis=1)

        @pl.when(pj < GD)
        def _d():
            finish(_dot(dsim[...], wdfcT[...]))

        @pl.when(pj >= GD)
        def _m():
            finish(_dot(msim[...], wmfcT[...]))

    @pl.when(pi == 1)
    def _pass1():
        row0 = pj * RB
        agg3 = _dot(adj[...], rhs_s[...])
        mu = agg3[:, :64]
        s2 = agg3[:, 64:128]
        g3 = agg3[:, 128:]
        sig = jnp.sqrt(jnp.where(s2 == 0, 1e-16, s2))
        graw = jnp.where(g3 == 0, 1e-16, g3)
        gam = jnp.sign(graw) * jnp.exp(jnp.log(jnp.abs(graw)) * (1.0 / 3.0))
        h0 = rhs_s[pl.ds(row0, RB), :64]
        h1 = _layer_epilogue(mu, h0, mu, sig, gam, wc1[...], w1t[...],
                             w1b[...], THETA1)
        h1_s[pl.ds(row0, RB), :] = h1
        mom_s[pl.ds(row0, RB), :] = jnp.concatenate([mu, sig, gam], axis=1)

    @pl.when(pi == 2)
    def _pass2():
        row0 = (G - 1 - pj) * RB
        agg = _dot(adj[...], h1_s[...])
        h0 = rhs_s[pl.ds(row0, RB), :64]
        mu = mom_s[pl.ds(row0, RB), :64]
        sig = mom_s[pl.ds(row0, RB), 64:128]
        gam = mom_s[pl.ds(row0, RB), 128:]
        h2 = _layer_epilogue(agg, h0, mu, sig, gam, wc2[...], w2t[...],
                             w2b[...], THETA2)
        hn = h2 * jax.lax.rsqrt(jnp.sum(h2 * h2, axis=1, keepdims=True))
        z = jnp.maximum(_dot(hn, w1T[...]) + b1[...], 0.0)
        logits = _dot(z, w2T[...]) + b2[...]
        mx = jnp.max(logits, axis=1, keepdims=True)
        f0 = logits - (mx + jnp.log(jnp.sum(jnp.exp(logits - mx), axis=1,
                                            keepdims=True)))
        feats = feats_s[pl.ds(row0, RB), :]
        Hd = _elu(_dot(f0, wd1aT[...]) + _dot(feats, wd1bT[...]) + bd1[...])
        Hm = _elu(_dot(f0, wm1aT[...]) + _dot(feats, wm1bT[...]) + bm1[...])
        rows = jax.lax.broadcasted_iota(jnp.int32, (RB, 1), 0) + row0
        H = jnp.where(rows < ND, Hd, Hm)
        ab_o[...] = _dot(H, wp[...]) + bp2[...]


def _fused(adj, d_sim, m_sim, topo, weights):
    full = lambda arr: pl.BlockSpec(arr.shape, lambda i, j: (0, 0))

    def adj_map(i, j):
        return (jnp.where(i == 0, 0, jnp.where(i == 1, j, G - 1 - j)), 0)

    def d_map(i, j):
        return (jnp.where(i == 0, jnp.minimum(j, GD - 1), GD - 1), 0)

    def m_map(i, j):
        return (jnp.where(i == 0, jnp.clip(j - GD, 0, G - GD - 1),
                          G - GD - 1), 0)

    def topo_map(i, j):
        return (jnp.where(i == 0, j, 0), 0)

    def ab_map(i, j):
        return (jnp.where(i == 2, G - 1 - j, j), 0)

    return pl.pallas_call(
        _fused_body,
        grid=(3, G),
        in_specs=[
            pl.BlockSpec((RB, N), adj_map),
            pl.BlockSpec((RB, ND), d_map),
            pl.BlockSpec((RB, NM), m_map),
            pl.BlockSpec((RB, 64), topo_map),
        ] + [full(w) for w in weights],
        out_specs=pl.BlockSpec((RB, 2), ab_map),
        out_shape=jax.ShapeDtypeStruct((N, 2), _f32),
        scratch_shapes=[
            pltpu.VMEM((N, 64), _f32),
            pltpu.VMEM((N, 192), _f32),
            pltpu.VMEM((N, 64), _f32),
            pltpu.VMEM((N, 192), _f32),
        ],
        compiler_params=pltpu.CompilerParams(
            dimension_semantics=("arbitrary", "arbitrary")),
    )(adj, d_sim, m_sim, topo, *weights)


_NW = 32
_BPW = B // _NW
_L = 16


@functools.partial(
    pl.kernel,
    mesh=plsc.VectorSubcoreMesh(core_axis_name="c", subcore_axis_name="s"),
    out_type=jax.ShapeDtypeStruct((B,), _f32),
    scratch_types=[
        pltpu.VMEM((_BPW,), jnp.int32),
        pltpu.VMEM((_BPW,), jnp.int32),
        pltpu.VMEM((_BPW,), _f32),
        pltpu.VMEM((_BPW,), _f32),
        pltpu.VMEM((_BPW,), _f32),
        pltpu.SemaphoreType.DMA,
    ],
)
def _pair_score(a_hbm, b_hbm, d_hbm, m_hbm, out_hbm, d_v, m_v, a_v, b_v, o_v,
                sem):
    wid = lax.axis_index("s") * 2 + lax.axis_index("c")
    base = wid * _BPW
    pltpu.sync_copy(d_hbm.at[pl.ds(base, _BPW)], d_v)
    pltpu.sync_copy(m_hbm.at[pl.ds(base, _BPW)], m_v)
    cp_a = pltpu.async_copy(a_hbm.at[d_v], a_v, sem)
    cp_b = pltpu.async_copy(b_hbm.at[m_v], b_v, sem)
    cp_a.wait()
    cp_b.wait()

    def body(j, carry):
        off = j * _L
        s = a_v[pl.ds(off, _L)] + b_v[pl.ds(off, _L)]
        o_v[pl.ds(off, _L)] = 1.0 / (1.0 + jnp.exp(-s))
        return carry

    lax.fori_loop(0, _BPW // _L, body, 0)
    pltpu.sync_copy(o_v, out_hbm.at[pl.ds(base, _BPW)])


def kernel(Topo, adj, d_sim, m_sim, params, diseases, mirnas):
    p = params
    r2 = lambda v: v.reshape(1, -1)
    wp = jnp.stack([p['Wp'][0, :64], p['Wp'][0, 64:]], axis=1)
    bp2 = jnp.stack([p['bp'][0], jnp.zeros((), _f32)]).reshape(1, 2)
    weights = (
        p['Wd_fc'].T, p['Wm_fc'].T,
        p['Wdown'][:, :64].T, p['Wdown'][:, 64:].T, r2(p['bdown']),
        p['Wfc0'].T, r2(p['bfc0']),
        p['conv_w'][0], p['conv_watt'][0][:64, :], p['conv_watt'][0][64:, :],
        p['conv_w'][1], p['conv_watt'][1][:64, :], p['conv_watt'][1][64:, :],
        p['W1'].T, r2(p['b1']), p['W2'].T, r2(p['b2']),
        p['Wd1'][:, :2].T, p['Wd1'][:, 2:].T, r2(p['bd1']),
        p['Wm1'][:, :2].T, p['Wm1'][:, 2:].T, r2(p['bm1']),
        wp, bp2,
    )
    ab = _fused(adj, d_sim, m_sim, Topo, weights)
    out = _pair_score(ab[:, 0], ab[:, 1], diseases, mirnas)
    return out.reshape(B, 1)


# speedup vs baseline: 1.1907x; 1.1907x over previous
import functools

import numpy as np
import jax
import jax.numpy as jnp
from jax import lax
from jax.experimental import pallas as pl
from jax.experimental.pallas import tpu as pltpu
from jax.experimental.pallas import tpu_sc as plsc

ND = 2000
NM = 3000
N = ND + NM
HID = 64
B = 16384
ALPHA = 0.1
BETA = 0.1
LAMDA = 0.5
THETA1 = float(np.log(LAMDA / 1.0 + 1.0))
THETA2 = float(np.log(LAMDA / 2.0 + 1.0))
RB = 1000
RB1 = 1000
G = N // RB
GD = ND // RB

_f32 = jnp.float32


def _dot(a, b):
    return jnp.dot(a, b, preferred_element_type=_f32)


def _elu(x):
    return jnp.where(x > 0, x, jnp.exp(jnp.minimum(x, 0.0)) - 1.0)


def _layer_epilogue(agg, h0, mu, sig, gam, w, watt_t, watt_b, theta):
    h_agg = (1.0 - ALPHA) * agg + ALPHA * h0
    h_i = theta * _dot(h_agg, w) + (1.0 - theta) * h_agg
    qb = _dot(h_i, watt_b)
    e_mu = _elu(_dot(mu, watt_t) + qb)
    e_si = _elu(_dot(sig, watt_t) + qb)
    e_ga = _elu(_dot(gam, watt_t) + qb)
    m = jnp.maximum(jnp.maximum(e_mu, e_si), e_ga)
    x_mu = jnp.exp(e_mu - m)
    x_si = jnp.exp(e_si - m)
    x_ga = jnp.exp(e_ga - m)
    h_mom = (mu * x_mu + sig * x_si + gam * x_ga) / (x_mu + x_si + x_ga)
    out = (1.0 - BETA) * h_i + BETA * h_mom
    rm = jnp.max(out, axis=1, keepdims=True)
    e = jnp.exp(out - rm)
    return e / jnp.sum(e, axis=1, keepdims=True)


def _stage1_body(sim, wfcT, topo, wdtT, wdfT, bdown, wfc0T, bfc0,
                 feats_o, rhs_o):
    f = _dot(sim[...], wfcT[...])
    x = _dot(topo[...], wdtT[...]) + _dot(f, wdfT[...]) + bdown[...]
    h0 = jnp.maximum(_dot(x, wfc0T[...]) + bfc0[...], 0.0)
    feats_o[...] = f
    rhs_o[...] = jnp.concatenate([h0, h0 * h0, h0 * h0 * h0], axis=1)


def _stage1(sim, wfcT, topo, wdtT, wdfT, bdown, wfc0T, bfc0):
    nrows, k = sim.shape
    grid = (nrows // RB1,)
    full = lambda arr: pl.BlockSpec(arr.shape, lambda i: (0, 0))
    return pl.pallas_call(
        _stage1_body,
        grid=grid,
        in_specs=[
            pl.BlockSpec((RB1, k), lambda i: (i, 0)),
            full(wfcT),
            pl.BlockSpec((RB1, 64), lambda i: (i, 0)),
            full(wdtT), full(wdfT), full(bdown), full(wfc0T), full(bfc0),
        ],
        out_specs=[
            pl.BlockSpec((RB1, 64), lambda i: (i, 0)),
            pl.BlockSpec((RB1, 192), lambda i: (i, 0)),
        ],
        out_shape=[
            jax.ShapeDtypeStruct((nrows, 64), _f32),
            jax.ShapeDtypeStruct((nrows, 192), _f32),
        ],
        compiler_params=pltpu.CompilerParams(
            dimension_semantics=("arbitrary",)),
    )(sim, wfcT, topo, wdtT, wdfT, bdown, wfc0T, bfc0)


def _layers_body(adj, rhs, feats, wc1, w1t, w1b, wc2, w2t, w2b,
                 w1T, b1, w2T, b2, wd1aT, wd1bT, bd1, wm1aT, wm1bT, bm1,
                 wp, bp2, ab_o, h1_s, mom_s):
    pi = pl.program_id(0)
    pj = pl.program_id(1)

    @pl.when(pi == 0)
    def _pass1():
        row0 = pj * RB
        agg3 = _dot(adj[...], rhs[...])
        mu = agg3[:, :64]
        s2 = agg3[:, 64:128]
        g3 = agg3[:, 128:]
        sig = jnp.sqrt(jnp.where(s2 == 0, 1e-16, s2))
        graw = jnp.where(g3 == 0, 1e-16, g3)
        gam = jnp.sign(graw) * jnp.exp(jnp.log(jnp.abs(graw)) * (1.0 / 3.0))
        h0 = rhs[pl.ds(row0, RB), :64]
        h1 = _layer_epilogue(mu, h0, mu, sig, gam, wc1[...], w1t[...],
                             w1b[...], THETA1)
        h1_s[pl.ds(row0, RB), :] = h1
        mom_s[pl.ds(row0, RB), :] = jnp.concatenate([mu, sig, gam], axis=1)

    @pl.when(pi == 1)
    def _pass2():
        row0 = (G - 1 - pj) * RB
        agg = _dot(adj[...], h1_s[...])
        h0 = rhs[pl.ds(row0, RB), :64]
        mu = mom_s[pl.ds(row0, RB), :64]
        sig = mom_s[pl.ds(row0, RB), 64:128]
        gam = mom_s[pl.ds(row0, RB), 128:]
        h2 = _layer_epilogue(agg, h0, mu, sig, gam, wc2[...], w2t[...],
                             w2b[...], THETA2)
        hn = h2 * jax.lax.rsqrt(jnp.sum(h2 * h2, axis=1, keepdims=True))
        z = jnp.maximum(_dot(hn, w1T[...]) + b1[...], 0.0)
        logits = _dot(z, w2T[...]) + b2[...]
        mx = jnp.max(logits, axis=1, keepdims=True)
        f0 = logits - (mx + jnp.log(jnp.sum(jnp.exp(logits - mx), axis=1,
                                            keepdims=True)))
        fb = feats[pl.ds(row0, RB), :]
        Hd = _elu(_dot(f0, wd1aT[...]) + _dot(fb, wd1bT[...]) + bd1[...])
        Hm = _elu(_dot(f0, wm1aT[...]) + _dot(fb, wm1bT[...]) + bm1[...])
        rows = jax.lax.broadcasted_iota(jnp.int32, (RB, 1), 0) + row0
        H = jnp.where(rows < ND, Hd, Hm)
        ab_o[...] = _dot(H, wp[...]) + bp2[...]


def _layers(adj, rhs, feats, weights):
    full = lambda arr: pl.BlockSpec(arr.shape, lambda i, j: (0, 0))

    def adj_map(i, j):
        return (jnp.where(i == 0, j, G - 1 - j), 0)

    return pl.pallas_call(
        _layers_body,
        grid=(2, G),
        in_specs=[
            pl.BlockSpec((RB, N), adj_map),
            full(rhs),
            full(feats),
        ] + [full(w) for w in weights],
        out_specs=pl.BlockSpec((RB, 2), adj_map),
        out_shape=jax.ShapeDtypeStruct((N, 2), _f32),
        scratch_shapes=[
            pltpu.VMEM((N, 64), _f32),
            pltpu.VMEM((N, 192), _f32),
        ],
        compiler_params=pltpu.CompilerParams(
            dimension_semantics=("arbitrary", "arbitrary")),
    )(adj, rhs, feats, *weights)


_NW = 32
_BPW = B // _NW
_L = 16


@functools.partial(
    pl.kernel,
    mesh=plsc.VectorSubcoreMesh(core_axis_name="c", subcore_axis_name="s"),
    out_type=jax.ShapeDtypeStruct((B,), _f32),
    scratch_types=[
        pltpu.VMEM((_BPW,), jnp.int32),
        pltpu.VMEM((_BPW,), jnp.int32),
        pltpu.VMEM((_BPW,), _f32),
        pltpu.VMEM((_BPW,), _f32),
        pltpu.VMEM((_BPW,), _f32),
        pltpu.SemaphoreType.DMA,
    ],
)
def _pair_score(a_hbm, b_hbm, d_hbm, m_hbm, out_hbm, d_v, m_v, a_v, b_v, o_v,
                sem):
    wid = lax.axis_index("s") * 2 + lax.axis_index("c")
    base = wid * _BPW
    pltpu.sync_copy(d_hbm.at[pl.ds(base, _BPW)], d_v)
    pltpu.sync_copy(m_hbm.at[pl.ds(base, _BPW)], m_v)
    cp_a = pltpu.async_copy(a_hbm.at[d_v], a_v, sem)
    cp_b = pltpu.async_copy(b_hbm.at[m_v], b_v, sem)
    cp_a.wait()
    cp_b.wait()

    def body(j, carry):
        off = j * _L
        s = a_v[pl.ds(off, _L)] + b_v[pl.ds(off, _L)]
        o_v[pl.ds(off, _L)] = 1.0 / (1.0 + jnp.exp(-s))
        return carry

    lax.fori_loop(0, _BPW // _L, body, 0)
    pltpu.sync_copy(o_v, out_hbm.at[pl.ds(base, _BPW)])


def kernel(Topo, adj, d_sim, m_sim, params, diseases, mirnas):
    p = params
    r2 = lambda v: v.reshape(1, -1)
    wp = jnp.stack([p['Wp'][0, :64], p['Wp'][0, 64:]], axis=1)
    bp2 = jnp.stack([p['bp'][0], jnp.zeros((), _f32)]).reshape(1, 2)
    wdtT = p['Wdown'][:, :64].T
    wdfT = p['Wdown'][:, 64:].T
    feats_d, rhs_d = _stage1(d_sim, p['Wd_fc'].T, Topo[:ND], wdtT, wdfT,
                             r2(p['bdown']), p['Wfc0'].T, r2(p['bfc0']))
    feats_m, rhs_m = _stage1(m_sim, p['Wm_fc'].T, Topo[ND:], wdtT, wdfT,
                             r2(p['bdown']), p['Wfc0'].T, r2(p['bfc0']))
    feats = jnp.concatenate([feats_d, feats_m], axis=0)
    rhs = jnp.concatenate([rhs_d, rhs_m], axis=0)
    weights = (
        p['conv_w'][0], p['conv_watt'][0][:64, :], p['conv_watt'][0][64:, :],
        p['conv_w'][1], p['conv_watt'][1][:64, :], p['conv_watt'][1][64:, :],
        p['W1'].T, r2(p['b1']), p['W2'].T, r2(p['b2']),
        p['Wd1'][:, :2].T, p['Wd1'][:, 2:].T, r2(p['bd1']),
        p['Wm1'][:, :2].T, p['Wm1'][:, 2:].T, r2(p['bm1']),
        wp, bp2,
    )
    ab = _layers(adj, rhs, feats, weights)
    out = _pair_score(ab[:, 0], ab[:, 1], diseases, mirnas)
    return out.reshape(B, 1)


# speedup vs baseline: 1.2835x; 1.0780x over previous
import functools

import numpy as np
import jax
import jax.numpy as jnp
from jax import lax
from jax.experimental import pallas as pl
from jax.experimental.pallas import tpu as pltpu
from jax.experimental.pallas import tpu_sc as plsc

ND = 2000
NM = 3000
N = ND + NM
HID = 64
B = 16384
ALPHA = 0.1
BETA = 0.1
LAMDA = 0.5
THETA1 = float(np.log(LAMDA / 1.0 + 1.0))
THETA2 = float(np.log(LAMDA / 2.0 + 1.0))
RB = 1000
RB1 = 1000
G = N // RB
GD = ND // RB

_f32 = jnp.float32


def _dot(a, b):
    return jnp.dot(a, b, preferred_element_type=_f32)


def _elu(x):
    return jnp.where(x > 0, x, jnp.exp(jnp.minimum(x, 0.0)) - 1.0)


def _layer_epilogue(agg, h0, mu, sig, gam, w, watt_t, watt_b, theta):
    h_agg = (1.0 - ALPHA) * agg + ALPHA * h0
    h_i = theta * _dot(h_agg, w) + (1.0 - theta) * h_agg
    qb = _dot(h_i, watt_b)
    e_mu = _elu(_dot(mu, watt_t) + qb)
    e_si = _elu(_dot(sig, watt_t) + qb)
    e_ga = _elu(_dot(gam, watt_t) + qb)
    m = jnp.maximum(jnp.maximum(e_mu, e_si), e_ga)
    x_mu = jnp.exp(e_mu - m)
    x_si = jnp.exp(e_si - m)
    x_ga = jnp.exp(e_ga - m)
    h_mom = (mu * x_mu + sig * x_si + gam * x_ga) / (x_mu + x_si + x_ga)
    out = (1.0 - BETA) * h_i + BETA * h_mom
    rm = jnp.max(out, axis=1, keepdims=True)
    e = jnp.exp(out - rm)
    return e / jnp.sum(e, axis=1, keepdims=True)


def _stage1_body(dsim, msim, topo, wdfcT, wmfcT, wdtT, wdfT, bdown,
                 wfc0T, bfc0, feats_o, rhs_o):
    pj = pl.program_id(0)

    def finish(f):
        x = _dot(topo[...], wdtT[...]) + _dot(f, wdfT[...]) + bdown[...]
        h0 = jnp.maximum(_dot(x, wfc0T[...]) + bfc0[...], 0.0)
        feats_o[...] = f
        rhs_o[...] = jnp.concatenate([h0, h0 * h0, h0 * h0 * h0], axis=1)

    @pl.when(pj < GD)
    def _d():
        finish(_dot(dsim[...], wdfcT[...]))

    @pl.when(pj >= GD)
    def _m():
        finish(_dot(msim[...], wmfcT[...]))


def _stage1(d_sim, m_sim, topo, wdfcT, wmfcT, wdtT, wdfT, bdown, wfc0T,
            bfc0):
    full = lambda arr: pl.BlockSpec(arr.shape, lambda i: (0, 0))
    return pl.pallas_call(
        _stage1_body,
        grid=(G,),
        in_specs=[
            pl.BlockSpec((RB, ND), lambda i: (jnp.minimum(i, GD - 1), 0)),
            pl.BlockSpec((RB, NM),
                         lambda i: (jnp.clip(i - GD, 0, G - GD - 1), 0)),
            pl.BlockSpec((RB, 64), lambda i: (i, 0)),
            full(wdfcT), full(wmfcT),
            full(wdtT), full(wdfT), full(bdown), full(wfc0T), full(bfc0),
        ],
        out_specs=[
            pl.BlockSpec((RB, 64), lambda i: (i, 0)),
            pl.BlockSpec((RB, 192), lambda i: (i, 0)),
        ],
        out_shape=[
            jax.ShapeDtypeStruct((N, 64), _f32),
            jax.ShapeDtypeStruct((N, 192), _f32),
        ],
        compiler_params=pltpu.CompilerParams(
            dimension_semantics=("arbitrary",)),
    )(d_sim, m_sim, topo, wdfcT, wmfcT, wdtT, wdfT, bdown, wfc0T, bfc0)


def _layers_body(adj, rhs, feats, wc1, w1t, w1b, wc2, w2t, w2b,
                 w1T, b1, w2T, b2, wd1aT, wd1bT, bd1, wm1aT, wm1bT, bm1,
                 wp, bp2, ab_o, h1_s, mom_s):
    pi = pl.program_id(0)
    pj = pl.program_id(1)

    @pl.when(pi == 0)
    def _pass1():
        row0 = pj * RB
        agg3 = _dot(adj[...], rhs[...])
        mu = agg3[:, :64]
        s2 = agg3[:, 64:128]
        g3 = agg3[:, 128:]
        sig = jnp.sqrt(jnp.where(s2 == 0, 1e-16, s2))
        graw = jnp.where(g3 == 0, 1e-16, g3)
        gam = jnp.sign(graw) * jnp.exp(jnp.log(jnp.abs(graw)) * (1.0 / 3.0))
        h0 = rhs[pl.ds(row0, RB), :64]
        h1 = _layer_epilogue(mu, h0, mu, sig, gam, wc1[...], w1t[...],
                             w1b[...], THETA1)
        h1_s[pl.ds(row0, RB), :] = h1
        mom_s[pl.ds(row0, RB), :] = jnp.concatenate([mu, sig, gam], axis=1)

    @pl.when(pi == 1)
    def _pass2():
        row0 = (G - 1 - pj) * RB
        agg = _dot(adj[...], h1_s[...])
        h0 = rhs[pl.ds(row0, RB), :64]
        mu = mom_s[pl.ds(row0, RB), :64]
        sig = mom_s[pl.ds(row0, RB), 64:128]
        gam = mom_s[pl.ds(row0, RB), 128:]
        h2 = _layer_epilogue(agg, h0, mu, sig, gam, wc2[...], w2t[...],
                             w2b[...], THETA2)
        hn = h2 * jax.lax.rsqrt(jnp.sum(h2 * h2, axis=1, keepdims=True))
        z = jnp.maximum(_dot(hn, w1T[...]) + b1[...], 0.0)
        logits = _dot(z, w2T[...]) + b2[...]
        mx = jnp.max(logits, axis=1, keepdims=True)
        f0 = logits - (mx + jnp.log(jnp.sum(jnp.exp(logits - mx), axis=1,
                                            keepdims=True)))
        fb = feats[pl.ds(row0, RB), :]
        Hd = _elu(_dot(f0, wd1aT[...]) + _dot(fb, wd1bT[...]) + bd1[...])
        Hm = _elu(_dot(f0, wm1aT[...]) + _dot(fb, wm1bT[...]) + bm1[...])
        rows = jax.lax.broadcasted_iota(jnp.int32, (RB, 1), 0) + row0
        H = jnp.where(rows < ND, Hd, Hm)
        ab_o[...] = _dot(H, wp[...]) + bp2[...]


def _layers(adj, rhs, feats, weights):
    full = lambda arr: pl.BlockSpec(arr.shape, lambda i, j: (0, 0))

    def adj_map(i, j):
        return (jnp.where(i == 0, j, G - 1 - j), 0)

    return pl.pallas_call(
        _layers_body,
        grid=(2, G),
        in_specs=[
            pl.BlockSpec((RB, N), adj_map),
            full(rhs),
            full(feats),
        ] + [full(w) for w in weights],
        out_specs=pl.BlockSpec((RB, 2), adj_map),
        out_shape=jax.ShapeDtypeStruct((N, 2), _f32),
        scratch_shapes=[
            pltpu.VMEM((N, 64), _f32),
            pltpu.VMEM((N, 192), _f32),
        ],
        compiler_params=pltpu.CompilerParams(
            dimension_semantics=("arbitrary", "arbitrary")),
    )(adj, rhs, feats, *weights)


_NW = 32
_BPW = B // _NW
_L = 16


@functools.partial(
    pl.kernel,
    mesh=plsc.VectorSubcoreMesh(core_axis_name="c", subcore_axis_name="s"),
    out_type=jax.ShapeDtypeStruct((B,), _f32),
    scratch_types=[
        pltpu.VMEM((_BPW,), jnp.int32),
        pltpu.VMEM((_BPW,), jnp.int32),
        pltpu.VMEM((_BPW,), _f32),
        pltpu.VMEM((_BPW,), _f32),
        pltpu.VMEM((_BPW,), _f32),
        pltpu.SemaphoreType.DMA,
    ],
)
def _pair_score(a_hbm, b_hbm, d_hbm, m_hbm, out_hbm, d_v, m_v, a_v, b_v, o_v,
                sem):
    wid = lax.axis_index("s") * 2 + lax.axis_index("c")
    base = wid * _BPW
    pltpu.sync_copy(d_hbm.at[pl.ds(base, _BPW)], d_v)
    pltpu.sync_copy(m_hbm.at[pl.ds(base, _BPW)], m_v)
    cp_a = pltpu.async_copy(a_hbm.at[d_v], a_v, sem)
    cp_b = pltpu.async_copy(b_hbm.at[m_v], b_v, sem)
    cp_a.wait()
    cp_b.wait()

    def body(j, carry):
        off = j * _L
        s = a_v[pl.ds(off, _L)] + b_v[pl.ds(off, _L)]
        o_v[pl.ds(off, _L)] = 1.0 / (1.0 + jnp.exp(-s))
        return carry

    lax.fori_loop(0, _BPW // _L, body, 0)
    pltpu.sync_copy(o_v, out_hbm.at[pl.ds(base, _BPW)])


def kernel(Topo, adj, d_sim, m_sim, params, diseases, mirnas):
    p = params
    r2 = lambda v: v.reshape(1, -1)
    wp = jnp.stack([p['Wp'][0, :64], p['Wp'][0, 64:]], axis=1)
    bp2 = jnp.stack([p['bp'][0], jnp.zeros((), _f32)]).reshape(1, 2)
    wdtT = p['Wdown'][:, :64].T
    wdfT = p['Wdown'][:, 64:].T
    feats, rhs = _stage1(d_sim, m_sim, Topo, p['Wd_fc'].T, p['Wm_fc'].T,
                         wdtT, wdfT, r2(p['bdown']), p['Wfc0'].T,
                         r2(p['bfc0']))
    weights = (
        p['conv_w'][0], p['conv_watt'][0][:64, :], p['conv_watt'][0][64:, :],
        p['conv_w'][1], p['conv_watt'][1][:64, :], p['conv_watt'][1][64:, :],
        p['W1'].T, r2(p['b1']), p['W2'].T, r2(p['b2']),
        p['Wd1'][:, :2].T, p['Wd1'][:, 2:].T, r2(p['bd1']),
        p['Wm1'][:, :2].T, p['Wm1'][:, 2:].T, r2(p['bm1']),
        wp, bp2,
    )
    ab = _layers(adj, rhs, feats, weights)
    out = _pair_score(ab[:, 0], ab[:, 1], diseases, mirnas)
    return out.reshape(B, 1)


# speedup vs baseline: 1.3686x; 1.0663x over previous
import functools

import numpy as np
import jax
import jax.numpy as jnp
from jax import lax
from jax.experimental import pallas as pl
from jax.experimental.pallas import tpu as pltpu
from jax.experimental.pallas import tpu_sc as plsc

ND = 2000
NM = 3000
N = ND + NM
HID = 64
B = 16384
ALPHA = 0.1
BETA = 0.1
LAMDA = 0.5
THETA1 = float(np.log(LAMDA / 1.0 + 1.0))
THETA2 = float(np.log(LAMDA / 2.0 + 1.0))
RB = 1000
RB1 = 1000
G = N // RB
GD = ND // RB

_f32 = jnp.float32


def _dot(a, b):
    return jnp.dot(a, b, preferred_element_type=_f32)


def _elu(x):
    return jnp.where(x > 0, x, jnp.exp(jnp.minimum(x, 0.0)) - 1.0)


def _layer_epilogue(agg, h0, mu, sig, gam, w, watt_t, watt_b, theta):
    h_agg = (1.0 - ALPHA) * agg + ALPHA * h0
    h_i = theta * _dot(h_agg, w) + (1.0 - theta) * h_agg
    qb = _dot(h_i, watt_b)
    e_mu = _elu(_dot(mu, watt_t) + qb)
    e_si = _elu(_dot(sig, watt_t) + qb)
    e_ga = _elu(_dot(gam, watt_t) + qb)
    m = jnp.maximum(jnp.maximum(e_mu, e_si), e_ga)
    x_mu = jnp.exp(e_mu - m)
    x_si = jnp.exp(e_si - m)
    x_ga = jnp.exp(e_ga - m)
    h_mom = (mu * x_mu + sig * x_si + gam * x_ga) / (x_mu + x_si + x_ga)
    out = (1.0 - BETA) * h_i + BETA * h_mom
    rm = jnp.max(out, axis=1, keepdims=True)
    e = jnp.exp(out - rm)
    return e / jnp.sum(e, axis=1, keepdims=True)


RBD = 200
RBM = 600
RB2 = 512
G2 = 10
NP = RB2 * G2


def _fused_body(adj, dsim, msim, topod, topom,
                wdfcT, wmfcT, wdtT, wdfT, bdown, wfc0T, bfc0,
                wc1, w1t, w1b, wc2, w2t, w2b,
                w1T, b1, w2T, b2, wd1aT, wd1bT, bd1, wm1aT, wm1bT, bm1,
                wp, bp2, ab_o, feats_s, rhs_s, h1_s, mom_s):
    pi = pl.program_id(0)
    pj = pl.program_id(1)

    def finish(f, topo, row0, rb):
        x = _dot(topo, wdtT[...]) + _dot(f, wdfT[...]) + bdown[...]
        h0 = jnp.maximum(_dot(x, wfc0T[...]) + bfc0[...], 0.0)
        feats_s[pl.ds(row0, rb), :] = f
        rhs_s[pl.ds(row0, rb), :] = jnp.concatenate(
            [h0, h0 * h0, h0 * h0 * h0], axis=1)

    @pl.when(pi == 0)
    def _p0():
        finish(_dot(dsim[...], wdfcT[...]), topod[...], pj * RBD, RBD)

    @pl.when((pi == 1) & (pj < 5))
    def _p1():
        finish(_dot(msim[...], wmfcT[...]), topom[...], ND + pj * RBM, RBM)

    @pl.when(pi == 2)
    def _p2():
        row0 = pj * RB2
        agg3 = _dot(adj[...], rhs_s[0:N, :])
        mu = agg3[:, :64]
        s2 = agg3[:, 64:128]
        g3 = agg3[:, 128:]
        sig = jnp.sqrt(jnp.where(s2 == 0, 1e-16, s2))
        graw = jnp.where(g3 == 0, 1e-16, g3)
        gam = jnp.sign(graw) * jnp.exp(jnp.log(jnp.abs(graw)) * (1.0 / 3.0))
        h0 = rhs_s[pl.ds(row0, RB2), :64]
        h1 = _layer_epilogue(mu, h0, mu, sig, gam, wc1[...], w1t[...],
                             w1b[...], THETA1)
        h1_s[pl.ds(row0, RB2), :] = h1
        mom_s[pl.ds(row0, RB2), :] = jnp.concatenate([mu, sig, gam], axis=1)

    @pl.when(pi == 3)
    def _p3():
        row0 = (G2 - 1 - pj) * RB2
        agg = _dot(adj[...], h1_s[0:N, :])
        h0 = rhs_s[pl.ds(row0, RB2), :64]
        mu = mom_s[pl.ds(row0, RB2), :64]
        sig = mom_s[pl.ds(row0, RB2), 64:128]
        gam = mom_s[pl.ds(row0, RB2), 128:]
        h2 = _layer_epilogue(agg, h0, mu, sig, gam, wc2[...], w2t[...],
                             w2b[...], THETA2)
        hn = h2 * jax.lax.rsqrt(jnp.sum(h2 * h2, axis=1, keepdims=True))
        z = jnp.maximum(_dot(hn, w1T[...]) + b1[...], 0.0)
        logits = _dot(z, w2T[...]) + b2[...]
        mx = jnp.max(logits, axis=1, keepdims=True)
        f0 = logits - (mx + jnp.log(jnp.sum(jnp.exp(logits - mx), axis=1,
                                            keepdims=True)))
        fb = feats_s[pl.ds(row0, RB2), :]
        Hd = _elu(_dot(f0, wd1aT[...]) + _dot(fb, wd1bT[...]) + bd1[...])
        Hm = _elu(_dot(f0, wm1aT[...]) + _dot(fb, wm1bT[...]) + bm1[...])
        rows = jax.lax.broadcasted_iota(jnp.int32, (RB2, 1), 0) + row0
        H = jnp.where(rows < ND, Hd, Hm)
        ab_o[...] = _dot(H, wp[...]) + bp2[...]


def _fused(adj, d_sim, m_sim, topo_d, topo_m, weights):
    full = lambda arr: pl.BlockSpec(arr.shape, lambda i, j: (0, 0))

    def adj_map(i, j):
        return (jnp.where(i < 2, 0,
                          jnp.where(i == 2, j, G2 - 1 - j)), 0)

    def d_map(i, j):
        return (jnp.where(i == 0, j, G2 - 1), 0)

    def m_map(i, j):
        return (jnp.where(i == 1, jnp.minimum(j, 4),
                          jnp.where(i == 0, 0, 4)), 0)

    def ab_map(i, j):
        return (jnp.where(i == 3, G2 - 1 - j, 0), 0)

    return pl.pallas_call(
        _fused_body,
        grid=(4, G2),
        in_specs=[
            pl.BlockSpec((RB2, N), adj_map),
            pl.BlockSpec((RBD, ND), d_map),
            pl.BlockSpec((RBM, NM), m_map),
            pl.BlockSpec((RBD, 64), d_map),
            pl.BlockSpec((RBM, 64), m_map),
        ] + [full(w) for w in weights],
        out_specs=pl.BlockSpec((RB2, 2), ab_map),
        out_shape=jax.ShapeDtypeStruct((N, 2), _f32),
        scratch_shapes=[
            pltpu.VMEM((NP, 64), _f32),
            pltpu.VMEM((NP, 192), _f32),
            pltpu.VMEM((NP, 64), _f32),
            pltpu.VMEM((NP, 192), _f32),
        ],
        compiler_params=pltpu.CompilerParams(
            dimension_semantics=("arbitrary", "arbitrary")),
    )(adj, d_sim, m_sim, topo_d, topo_m, *weights)


_NW = 32
_BPW = B // _NW
_L = 16


@functools.partial(
    pl.kernel,
    mesh=plsc.VectorSubcoreMesh(core_axis_name="c", subcore_axis_name="s"),
    out_type=jax.ShapeDtypeStruct((B,), _f32),
    scratch_types=[
        pltpu.VMEM((_BPW,), jnp.int32),
        pltpu.VMEM((_BPW,), jnp.int32),
        pltpu.VMEM((_BPW,), _f32),
        pltpu.VMEM((_BPW,), _f32),
        pltpu.VMEM((_BPW,), _f32),
        pltpu.SemaphoreType.DMA,
    ],
)
def _pair_score(a_hbm, b_hbm, d_hbm, m_hbm, out_hbm, d_v, m_v, a_v, b_v, o_v,
                sem):
    wid = lax.axis_index("s") * 2 + lax.axis_index("c")
    base = wid * _BPW
    pltpu.sync_copy(d_hbm.at[pl.ds(base, _BPW)], d_v)
    pltpu.sync_copy(m_hbm.at[pl.ds(base, _BPW)], m_v)
    cp_a = pltpu.async_copy(a_hbm.at[d_v], a_v, sem)
    cp_b = pltpu.async_copy(b_hbm.at[m_v], b_v, sem)
    cp_a.wait()
    cp_b.wait()

    def body(j, carry):
        off = j * _L
        s = a_v[pl.ds(off, _L)] + b_v[pl.ds(off, _L)]
        o_v[pl.ds(off, _L)] = 1.0 / (1.0 + jnp.exp(-s))
        return carry

    lax.fori_loop(0, _BPW // _L, body, 0)
    pltpu.sync_copy(o_v, out_hbm.at[pl.ds(base, _BPW)])


def kernel(Topo, adj, d_sim, m_sim, params, diseases, mirnas):
    p = params
    r2 = lambda v: v.reshape(1, -1)
    wp = jnp.stack([p['Wp'][0, :64], p['Wp'][0, 64:]], axis=1)
    bp2 = jnp.stack([p['bp'][0], jnp.zeros((), _f32)]).reshape(1, 2)
    weights = (
        p['Wd_fc'].T, p['Wm_fc'].T,
        p['Wdown'][:, :64].T, p['Wdown'][:, 64:].T, r2(p['bdown']),
        p['Wfc0'].T, r2(p['bfc0']),
        p['conv_w'][0], p['conv_watt'][0][:64, :], p['conv_watt'][0][64:, :],
        p['conv_w'][1], p['conv_watt'][1][:64, :], p['conv_watt'][1][64:, :],
        p['W1'].T, r2(p['b1']), p['W2'].T, r2(p['b2']),
        p['Wd1'][:, :2].T, p['Wd1'][:, 2:].T, r2(p['bd1']),
        p['Wm1'][:, :2].T, p['Wm1'][:, 2:].T, r2(p['bm1']),
        wp, bp2,
    )
    ab = _fused(adj, d_sim, m_sim, Topo[:ND], Topo[ND:], weights)
    out = _pair_score(ab[:, 0], ab[:, 1], diseases, mirnas)
    return out.reshape(B, 1)
